# initial kernel scaffold (unmeasured)
import jax
import jax.numpy as jnp
from jax import lax
from jax.experimental import pallas as pl
from jax.experimental.pallas import tpu as pltpu

N_DEV = 16


def kernel(A, B):
    m_per, k = A.shape
    _, n = B.shape

    def body(a_ref, b_ref, out_ref, comm_ref, c_ref, send_sems, recv_sems, out_sem):
        my = lax.axis_index("i")
        left = (my - 1) % N_DEV
        right = (my + 1) % N_DEV

        barrier_sem = pltpu.get_barrier_semaphore()
        for nbr in (left, right):
            pl.semaphore_signal(
                barrier_sem, inc=1,
                device_id=(nbr,), device_id_type=pl.DeviceIdType.MESH,
            )
        pl.semaphore_wait(barrier_sem, 2)

        comm_ref[0] = a_ref[...]

        for h in range(N_DEV):
            slot = h % 2
            if h > 0:
                pass
            if h < N_DEV - 1:
                rdma = pltpu.make_async_remote_copy(
                    src_ref=comm_ref.at[slot],
                    dst_ref=comm_ref.at[(h + 1) % 2],
                    send_sem=send_sems.at[slot],
                    recv_sem=recv_sems.at[(h + 1) % 2],
                    device_id=(right,),
                    device_id_type=pl.DeviceIdType.MESH,
                )
                rdma.start()

            origin = (my - h) % N_DEV
            c_ref[...] = jnp.dot(
                comm_ref[slot], b_ref[...],
                preferred_element_type=jnp.float32,
            )
            copy = pltpu.make_async_copy(
                c_ref,
                out_ref.at[pl.ds(origin * m_per, m_per), :],
                out_sem,
            )
            copy.start()
            copy.wait()

            if h < N_DEV - 1:
                rdma.wait()

    return pl.pallas_call(
        body,
        out_shape=jax.ShapeDtypeStruct((N_DEV * m_per, n), jnp.float32),
        in_specs=[
            pl.BlockSpec(memory_space=pltpu.VMEM),
            pl.BlockSpec(memory_space=pltpu.VMEM),
        ],
        out_specs=pl.BlockSpec(memory_space=pltpu.ANY),
        scratch_shapes=[
            pltpu.VMEM((2, m_per, k), jnp.float32),
            pltpu.VMEM((m_per, n), jnp.float32),
            pltpu.SemaphoreType.DMA((2,)),
            pltpu.SemaphoreType.DMA((2,)),
            pltpu.SemaphoreType.DMA,
        ],
        compiler_params=pltpu.CompilerParams(collective_id=0),
    )(A, B)


# baseline (device time: 900468 ns/iter reference)
import jax
import jax.numpy as jnp
from jax import lax
from jax.experimental import pallas as pl
from jax.experimental.pallas import tpu as pltpu

N_DEV = 16


def kernel(A, B):
    m_per, k = A.shape
    _, n = B.shape

    def body(a_ref, b_ref, out_ref, comm_ref, c_ref, send_sems, recv_sems, out_sem):
        my = lax.axis_index("i")
        left = (my - 1) % N_DEV
        right = (my + 1) % N_DEV

        barrier_sem = pltpu.get_barrier_semaphore()
        for nbr in (left, right):
            pl.semaphore_signal(
                barrier_sem, inc=1,
                device_id=(nbr,), device_id_type=pl.DeviceIdType.MESH,
            )
        pl.semaphore_wait(barrier_sem, 2)

        comm_ref[0] = a_ref[...]

        for h in range(N_DEV):
            slot = h % 2
            if h > 0:
                pass
            if h < N_DEV - 1:
                rdma = pltpu.make_async_remote_copy(
                    src_ref=comm_ref.at[slot],
                    dst_ref=comm_ref.at[(h + 1) % 2],
                    send_sem=send_sems.at[slot],
                    recv_sem=recv_sems.at[(h + 1) % 2],
                    device_id=(right,),
                    device_id_type=pl.DeviceIdType.MESH,
                )
                rdma.start()

            origin = (my - h) % N_DEV
            c_ref[...] = jnp.dot(
                comm_ref[slot], b_ref[...],
                preferred_element_type=jnp.float32,
            )
            copy = pltpu.make_async_copy(
                c_ref,
                out_ref.at[pl.ds(origin * m_per, m_per), :],
                out_sem,
            )
            copy.start()
            copy.wait()

            if h < N_DEV - 1:
                rdma.wait()

    return pl.pallas_call(
        body,
        out_shape=jax.ShapeDtypeStruct((N_DEV * m_per, n), jnp.float32),
        in_specs=[
            pl.BlockSpec(memory_space=pltpu.VMEM),
            pl.BlockSpec(memory_space=pltpu.VMEM),
        ],
        out_specs=pl.BlockSpec(memory_space=pl.ANY),
        scratch_shapes=[
            pltpu.VMEM((2, m_per, k), jnp.float32),
            pltpu.VMEM((m_per, n), jnp.float32),
            pltpu.SemaphoreType.DMA((2,)),
            pltpu.SemaphoreType.DMA((2,)),
            pltpu.SemaphoreType.DMA,
        ],
        compiler_params=pltpu.CompilerParams(collective_id=0),
    )(A, B)


# device time: 545950 ns/iter; 1.6494x vs baseline; 1.6494x over previous
import jax
import jax.numpy as jnp
from jax import lax
from jax.experimental import pallas as pl
from jax.experimental.pallas import tpu as pltpu

N_DEV = 16
R_HOPS = N_DEV // 2
L_HOPS = N_DEV - 1 - R_HOPS


def kernel(A, B):
    m_per, k = A.shape
    _, n = B.shape

    def body(a_ref, b_ref, out_ref, comm_r, comm_l, c_ref,
             send_r, recv_r, send_l, recv_l, out_sems):
        my = lax.axis_index("i")
        left = (my - 1) % N_DEV
        right = (my + 1) % N_DEV

        barrier_sem = pltpu.get_barrier_semaphore()
        for nbr in (left, right):
            pl.semaphore_signal(
                barrier_sem, inc=1,
                device_id=(nbr,), device_id_type=pl.DeviceIdType.MESH,
            )
        pl.semaphore_wait(barrier_sem, 2)

        comm_r[0] = a_ref[...]
        comm_l[0] = a_ref[...]

        pending = [None, None]
        count = [0]

        def emit_block(origin, chunk):
            s = count[0] % 2
            count[0] += 1
            if pending[s] is not None:
                pending[s].wait()
            c_ref[s] = jnp.dot(
                chunk, b_ref[...], preferred_element_type=jnp.float32
            )
            cp = pltpu.make_async_copy(
                c_ref.at[s],
                out_ref.at[pl.ds(origin * m_per, m_per), :],
                out_sems.at[s],
            )
            cp.start()
            pending[s] = cp

        for h in range(1, R_HOPS + 1):
            ss = (h - 1) % 2
            rs = h % 2
            rdma_r = pltpu.make_async_remote_copy(
                src_ref=comm_r.at[ss],
                dst_ref=comm_r.at[rs],
                send_sem=send_r.at[ss],
                recv_sem=recv_r.at[rs],
                device_id=(right,),
                device_id_type=pl.DeviceIdType.MESH,
            )
            rdma_r.start()
            rdma_l = None
            if h <= L_HOPS:
                rdma_l = pltpu.make_async_remote_copy(
                    src_ref=comm_l.at[ss],
                    dst_ref=comm_l.at[rs],
                    send_sem=send_l.at[ss],
                    recv_sem=recv_l.at[rs],
                    device_id=(left,),
                    device_id_type=pl.DeviceIdType.MESH,
                )
                rdma_l.start()

            if h == 1:
                emit_block(my, a_ref[...])
            else:
                emit_block((my - (h - 1)) % N_DEV, comm_r[ss])
                emit_block((my + (h - 1)) % N_DEV, comm_l[ss])

            rdma_r.wait()
            if rdma_l is not None:
                rdma_l.wait()

        emit_block((my - R_HOPS) % N_DEV, comm_r[R_HOPS % 2])
        for p in pending:
            if p is not None:
                p.wait()

    return pl.pallas_call(
        body,
        out_shape=jax.ShapeDtypeStruct((N_DEV * m_per, n), jnp.float32),
        in_specs=[
            pl.BlockSpec(memory_space=pltpu.VMEM),
            pl.BlockSpec(memory_space=pltpu.VMEM),
        ],
        out_specs=pl.BlockSpec(memory_space=pl.ANY),
        scratch_shapes=[
            pltpu.VMEM((2, m_per, k), jnp.float32),
            pltpu.VMEM((2, m_per, k), jnp.float32),
            pltpu.VMEM((2, m_per, n), jnp.float32),
            pltpu.SemaphoreType.DMA((2,)),
            pltpu.SemaphoreType.DMA((2,)),
            pltpu.SemaphoreType.DMA((2,)),
            pltpu.SemaphoreType.DMA((2,)),
            pltpu.SemaphoreType.DMA((2,)),
        ],
        compiler_params=pltpu.CompilerParams(
            collective_id=0,
            vmem_limit_bytes=60 * 1024 * 1024,
        ),
    )(A, B)


# device time: 545128 ns/iter; 1.6518x vs baseline; 1.0015x over previous
import jax
import jax.numpy as jnp
from jax import lax
from jax.experimental import pallas as pl
from jax.experimental.pallas import tpu as pltpu

N_DEV = 16
R_HOPS = N_DEV // 2
L_HOPS = N_DEV - 1 - R_HOPS


def kernel(A, B):
    m_per, k = A.shape
    _, n = B.shape

    def body(a_ref, b_ref, out_ref, comm_r, comm_l, c_ref,
             send_r, recv_r, send_l, recv_l, out_sems):
        my = lax.axis_index("i")
        left = (my - 1) % N_DEV
        right = (my + 1) % N_DEV

        barrier_sem = pltpu.get_barrier_semaphore()
        for nbr in (left, right):
            pl.semaphore_signal(
                barrier_sem, inc=1,
                device_id=(nbr,), device_id_type=pl.DeviceIdType.MESH,
            )
        pl.semaphore_wait(barrier_sem, 2)

        pending = [None, None]
        count = [0]

        def emit_block(origin, chunk):
            s = count[0] % 2
            count[0] += 1
            if pending[s] is not None:
                pending[s].wait()
            c_ref[s] = jnp.dot(
                chunk, b_ref[...], preferred_element_type=jnp.float32
            )
            cp = pltpu.make_async_copy(
                c_ref.at[s],
                out_ref.at[pl.ds(origin * m_per, m_per), :],
                out_sems.at[s],
            )
            cp.start()
            pending[s] = cp

        for h in range(1, R_HOPS + 1):
            ss = (h - 1) % 2
            rs = h % 2
            src_r = a_ref if h == 1 else comm_r.at[ss]
            src_l = a_ref if h == 1 else comm_l.at[ss]
            rdma_r = pltpu.make_async_remote_copy(
                src_ref=src_r,
                dst_ref=comm_r.at[rs],
                send_sem=send_r.at[ss],
                recv_sem=recv_r.at[rs],
                device_id=(right,),
                device_id_type=pl.DeviceIdType.MESH,
            )
            rdma_r.start()
            rdma_l = None
            if h <= L_HOPS:
                rdma_l = pltpu.make_async_remote_copy(
                    src_ref=src_l,
                    dst_ref=comm_l.at[rs],
                    send_sem=send_l.at[ss],
                    recv_sem=recv_l.at[rs],
                    device_id=(left,),
                    device_id_type=pl.DeviceIdType.MESH,
                )
                rdma_l.start()

            if h == 1:
                emit_block(my, a_ref[...])
            else:
                emit_block((my - (h - 1)) % N_DEV, comm_r[ss])
                emit_block((my + (h - 1)) % N_DEV, comm_l[ss])

            rdma_r.wait()
            if rdma_l is not None:
                rdma_l.wait()

        emit_block((my - R_HOPS) % N_DEV, comm_r[R_HOPS % 2])
        for p in pending:
            if p is not None:
                p.wait()

    out = pl.pallas_call(
        body,
        out_shape=jax.ShapeDtypeStruct((N_DEV * m_per, n), jnp.float32),
        in_specs=[
            pl.BlockSpec(memory_space=pltpu.VMEM),
            pl.BlockSpec(memory_space=pltpu.VMEM),
        ],
        out_specs=pl.BlockSpec(memory_space=pl.ANY),
        scratch_shapes=[
            pltpu.VMEM((2, m_per, k), jnp.float32),
            pltpu.VMEM((2, m_per, k), jnp.float32),
            pltpu.VMEM((2, m_per, n), jnp.float32),
            pltpu.SemaphoreType.DMA((2,)),
            pltpu.SemaphoreType.DMA((2,)),
            pltpu.SemaphoreType.DMA((2,)),
            pltpu.SemaphoreType.DMA((2,)),
            pltpu.SemaphoreType.DMA((2,)),
        ],
        compiler_params=pltpu.CompilerParams(
            collective_id=0,
            vmem_limit_bytes=60 * 1024 * 1024,
        ),
    )(A, B)
    return out


# device time: 343017 ns/iter; 2.6251x vs baseline; 1.5892x over previous
import jax
import jax.numpy as jnp
from jax import lax
from jax.experimental import pallas as pl
from jax.experimental.pallas import tpu as pltpu

N_DEV = 16
R_HOPS = N_DEV // 2
L_HOPS = N_DEV - 1 - R_HOPS


def kernel(A, B):
    m_per, k = A.shape
    _, n = B.shape

    def body(a_ref, b_ref, out_ref, a_bf, b_bf, comm_r, comm_l, c_ref,
             send_r, recv_r, send_l, recv_l, out_sems):
        my = lax.axis_index("i")
        left = (my - 1) % N_DEV
        right = (my + 1) % N_DEV

        a_bf[...] = a_ref[...].astype(jnp.bfloat16)
        b_bf[...] = b_ref[...].astype(jnp.bfloat16)

        barrier_sem = pltpu.get_barrier_semaphore()
        for nbr in (left, right):
            pl.semaphore_signal(
                barrier_sem, inc=1,
                device_id=(nbr,), device_id_type=pl.DeviceIdType.MESH,
            )
        pl.semaphore_wait(barrier_sem, 2)

        pending = [None, None]
        count = [0]

        def emit_block(origin, chunk):
            s = count[0] % 2
            count[0] += 1
            if pending[s] is not None:
                pending[s].wait()
            c_ref[s] = jnp.dot(
                chunk, b_bf[...], preferred_element_type=jnp.float32
            )
            cp = pltpu.make_async_copy(
                c_ref.at[s],
                out_ref.at[pl.ds(origin * m_per, m_per), :],
                out_sems.at[s],
            )
            cp.start()
            pending[s] = cp

        for h in range(1, R_HOPS + 1):
            ss = (h - 1) % 2
            rs = h % 2
            src_r = a_bf if h == 1 else comm_r.at[ss]
            src_l = a_bf if h == 1 else comm_l.at[ss]
            rdma_r = pltpu.make_async_remote_copy(
                src_ref=src_r,
                dst_ref=comm_r.at[rs],
                send_sem=send_r.at[ss],
                recv_sem=recv_r.at[rs],
                device_id=(right,),
                device_id_type=pl.DeviceIdType.MESH,
            )
            rdma_r.start()
            rdma_l = None
            if h <= L_HOPS:
                rdma_l = pltpu.make_async_remote_copy(
                    src_ref=src_l,
                    dst_ref=comm_l.at[rs],
                    send_sem=send_l.at[ss],
                    recv_sem=recv_l.at[rs],
                    device_id=(left,),
                    device_id_type=pl.DeviceIdType.MESH,
                )
                rdma_l.start()

            if h == 1:
                emit_block(my, a_bf[...])
            else:
                emit_block((my - (h - 1)) % N_DEV, comm_r[ss])
                emit_block((my + (h - 1)) % N_DEV, comm_l[ss])

            rdma_r.wait()
            if rdma_l is not None:
                rdma_l.wait()

        emit_block((my - R_HOPS) % N_DEV, comm_r[R_HOPS % 2])
        for p in pending:
            if p is not None:
                p.wait()

    out = pl.pallas_call(
        body,
        out_shape=jax.ShapeDtypeStruct((N_DEV * m_per, n), jnp.float32),
        in_specs=[
            pl.BlockSpec(memory_space=pltpu.VMEM),
            pl.BlockSpec(memory_space=pltpu.VMEM),
        ],
        out_specs=pl.BlockSpec(memory_space=pl.ANY),
        scratch_shapes=[
            pltpu.VMEM((m_per, k), jnp.bfloat16),
            pltpu.VMEM((k, n), jnp.bfloat16),
            pltpu.VMEM((2, m_per, k), jnp.bfloat16),
            pltpu.VMEM((2, m_per, k), jnp.bfloat16),
            pltpu.VMEM((2, m_per, n), jnp.float32),
            pltpu.SemaphoreType.DMA((2,)),
            pltpu.SemaphoreType.DMA((2,)),
            pltpu.SemaphoreType.DMA((2,)),
            pltpu.SemaphoreType.DMA((2,)),
            pltpu.SemaphoreType.DMA((2,)),
        ],
        compiler_params=pltpu.CompilerParams(
            collective_id=0,
            vmem_limit_bytes=60 * 1024 * 1024,
        ),
    )(A, B)
    return out


# device time: 320651 ns/iter; 2.8082x vs baseline; 1.0698x over previous
import jax
import jax.numpy as jnp
from jax import lax
from jax.experimental import pallas as pl
from jax.experimental.pallas import tpu as pltpu

N_DEV = 16
R_HOPS = N_DEV // 2
L_HOPS = N_DEV - 1 - R_HOPS


def kernel(A, B):
    m_per, k = A.shape
    _, n = B.shape

    def body(a_ref, b_ref, out_ref, a_bf, b_bf, comm_r, comm_l, c_ref,
             send_r, recv_r, send_l, recv_l, out_sems):
        my = lax.axis_index("i")
        left = (my - 1) % N_DEV
        right = (my + 1) % N_DEV

        a_bf[...] = a_ref[...].astype(jnp.bfloat16)
        b_bf[...] = b_ref[...].astype(jnp.bfloat16)

        barrier_sem = pltpu.get_barrier_semaphore()
        for nbr in (left, right):
            pl.semaphore_signal(
                barrier_sem, inc=1,
                device_id=(nbr,), device_id_type=pl.DeviceIdType.MESH,
            )
        pl.semaphore_wait(barrier_sem, 2)

        pending = [None, None]
        count = [0]

        def emit_block(origin, chunk):
            s = count[0] % 2
            count[0] += 1
            if pending[s] is not None:
                pending[s].wait()
            c_ref[s] = jnp.dot(
                chunk, b_bf[...], preferred_element_type=jnp.float32
            )
            cp = pltpu.make_async_copy(
                c_ref.at[s],
                out_ref.at[pl.ds(origin * m_per, m_per), :],
                out_sems.at[s],
            )
            cp.start()
            pending[s] = cp

        S = 2
        m_sub = m_per // S

        def msg(h, j, comm, send_s, recv_s, target):
            ss = (h - 1) % 2
            rs = h % 2
            rows = pl.ds(j * m_sub, m_sub)
            src = a_bf.at[rows, :] if h == 1 else comm.at[ss, rows, :]
            return pltpu.make_async_remote_copy(
                src_ref=src,
                dst_ref=comm.at[rs, rows, :],
                send_sem=send_s.at[ss, j],
                recv_sem=recv_s.at[rs, j],
                device_id=(target,),
                device_id_type=pl.DeviceIdType.MESH,
            )

        def msg_r(h, j):
            return msg(h, j, comm_r, send_r, recv_r, right)

        def msg_l(h, j):
            return msg(h, j, comm_l, send_l, recv_l, left)

        for h in range(1, R_HOPS + 1):
            ss = (h - 1) % 2
            for j in range(S):
                if h >= 2:
                    msg_r(h - 1, j).wait_recv()
                if h >= 3:
                    msg_r(h - 2, j).wait_send()
                msg_r(h, j).start()
                if 2 <= h <= L_HOPS + 1:
                    msg_l(h - 1, j).wait_recv()
                if 3 <= h <= L_HOPS + 2:
                    msg_l(h - 2, j).wait_send()
                if h <= L_HOPS:
                    msg_l(h, j).start()

            if h == 1:
                emit_block(my, a_bf[...])
            else:
                emit_block((my - (h - 1)) % N_DEV, comm_r[ss])
                if h - 1 <= L_HOPS:
                    emit_block((my + (h - 1)) % N_DEV, comm_l[ss])

        for j in range(S):
            msg_r(R_HOPS, j).wait_recv()
            msg_r(R_HOPS - 1, j).wait_send()
            msg_r(R_HOPS, j).wait_send()
            msg_l(L_HOPS, j).wait_send()
        emit_block((my - R_HOPS) % N_DEV, comm_r[R_HOPS % 2])
        for p in pending:
            if p is not None:
                p.wait()

    out = pl.pallas_call(
        body,
        out_shape=jax.ShapeDtypeStruct((N_DEV * m_per, n), jnp.float32),
        in_specs=[
            pl.BlockSpec(memory_space=pltpu.VMEM),
            pl.BlockSpec(memory_space=pltpu.VMEM),
        ],
        out_specs=pl.BlockSpec(memory_space=pl.ANY),
        scratch_shapes=[
            pltpu.VMEM((m_per, k), jnp.bfloat16),
            pltpu.VMEM((k, n), jnp.bfloat16),
            pltpu.VMEM((2, m_per, k), jnp.bfloat16),
            pltpu.VMEM((2, m_per, k), jnp.bfloat16),
            pltpu.VMEM((2, m_per, n), jnp.float32),
            pltpu.SemaphoreType.DMA((2, 2)),
            pltpu.SemaphoreType.DMA((2, 2)),
            pltpu.SemaphoreType.DMA((2, 2)),
            pltpu.SemaphoreType.DMA((2, 2)),
            pltpu.SemaphoreType.DMA((2,)),
        ],
        compiler_params=pltpu.CompilerParams(
            collective_id=0,
            vmem_limit_bytes=60 * 1024 * 1024,
        ),
    )(A, B)
    return out


# device time: 315522 ns/iter; 2.8539x vs baseline; 1.0163x over previous
import jax
import jax.numpy as jnp
from jax import lax
from jax.experimental import pallas as pl
from jax.experimental.pallas import tpu as pltpu

N_DEV = 16
R_HOPS = N_DEV // 2
L_HOPS = N_DEV - 1 - R_HOPS


def kernel(A, B):
    m_per, k = A.shape
    _, n = B.shape

    def body(a_ref, b_ref, out_ref, a_bf, b_bf, comm_r, comm_l, c_ref,
             send_r, recv_r, send_l, recv_l, out_sems):
        my = lax.axis_index("i")
        left = (my - 1) % N_DEV
        right = (my + 1) % N_DEV

        a_bf[...] = a_ref[...].astype(jnp.bfloat16)
        b_bf[...] = b_ref[...].astype(jnp.bfloat16)

        barrier_sem = pltpu.get_barrier_semaphore()
        for nbr in (left, right):
            pl.semaphore_signal(
                barrier_sem, inc=1,
                device_id=(nbr,), device_id_type=pl.DeviceIdType.MESH,
            )
        pl.semaphore_wait(barrier_sem, 2)

        pending = [None, None]
        count = [0]

        def emit_block(origin, chunk):
            s = count[0] % 2
            count[0] += 1
            if pending[s] is not None:
                pending[s].wait()
            c_ref[s] = jnp.dot(
                chunk, b_bf[...], preferred_element_type=jnp.float32
            )
            cp = pltpu.make_async_copy(
                c_ref.at[s],
                out_ref.at[pl.ds(origin * m_per, m_per), :],
                out_sems.at[s],
            )
            cp.start()
            pending[s] = cp

        S = 4
        m_sub = m_per // S

        def msg(h, j, comm, send_s, recv_s, target):
            ss = (h - 1) % 2
            rs = h % 2
            rows = pl.ds(j * m_sub, m_sub)
            src = a_bf.at[rows, :] if h == 1 else comm.at[ss, rows, :]
            return pltpu.make_async_remote_copy(
                src_ref=src,
                dst_ref=comm.at[rs, rows, :],
                send_sem=send_s.at[ss, j],
                recv_sem=recv_s.at[rs, j],
                device_id=(target,),
                device_id_type=pl.DeviceIdType.MESH,
            )

        def msg_r(h, j):
            return msg(h, j, comm_r, send_r, recv_r, right)

        def msg_l(h, j):
            return msg(h, j, comm_l, send_l, recv_l, left)

        for h in range(1, R_HOPS + 1):
            ss = (h - 1) % 2
            for j in range(S):
                if h >= 2:
                    msg_r(h - 1, j).wait_recv()
                if h >= 3:
                    msg_r(h - 2, j).wait_send()
                msg_r(h, j).start()
                if 2 <= h <= L_HOPS + 1:
                    msg_l(h - 1, j).wait_recv()
                if 3 <= h <= L_HOPS + 2:
                    msg_l(h - 2, j).wait_send()
                if h <= L_HOPS:
                    msg_l(h, j).start()

            if h == 1:
                emit_block(my, a_bf[...])
            else:
                emit_block((my - (h - 1)) % N_DEV, comm_r[ss])
                if h - 1 <= L_HOPS:
                    emit_block((my + (h - 1)) % N_DEV, comm_l[ss])

        f_origin = (my - R_HOPS) % N_DEV
        fs = R_HOPS % 2
        for j in range(S):
            msg_r(R_HOPS, j).wait_recv()
            s = count[0] % 2
            count[0] += 1
            if pending[s] is not None:
                pending[s].wait()
            c_ref[s, :m_sub, :] = jnp.dot(
                comm_r[fs, pl.ds(j * m_sub, m_sub), :], b_bf[...],
                preferred_element_type=jnp.float32,
            )
            cp = pltpu.make_async_copy(
                c_ref.at[s, pl.ds(0, m_sub), :],
                out_ref.at[pl.ds(f_origin * m_per + j * m_sub, m_sub), :],
                out_sems.at[s],
            )
            cp.start()
            pending[s] = cp
        for j in range(S):
            msg_r(R_HOPS - 1, j).wait_send()
            msg_r(R_HOPS, j).wait_send()
            msg_l(L_HOPS, j).wait_send()
        for p in pending:
            if p is not None:
                p.wait()

    out = pl.pallas_call(
        body,
        out_shape=jax.ShapeDtypeStruct((N_DEV * m_per, n), jnp.float32),
        in_specs=[
            pl.BlockSpec(memory_space=pltpu.VMEM),
            pl.BlockSpec(memory_space=pltpu.VMEM),
        ],
        out_specs=pl.BlockSpec(memory_space=pltpu.MemorySpace.HBM),
        scratch_shapes=[
            pltpu.VMEM((m_per, k), jnp.bfloat16),
            pltpu.VMEM((k, n), jnp.bfloat16),
            pltpu.VMEM((2, m_per, k), jnp.bfloat16),
            pltpu.VMEM((2, m_per, k), jnp.bfloat16),
            pltpu.VMEM((2, m_per, n), jnp.float32),
            pltpu.SemaphoreType.DMA((2, 4)),
            pltpu.SemaphoreType.DMA((2, 4)),
            pltpu.SemaphoreType.DMA((2, 4)),
            pltpu.SemaphoreType.DMA((2, 4)),
            pltpu.SemaphoreType.DMA((2,)),
        ],
        compiler_params=pltpu.CompilerParams(
            collective_id=0,
            vmem_limit_bytes=60 * 1024 * 1024,
        ),
    )(A, B)
    return out
